# Initial kernel scaffold; baseline (speedup 1.0000x reference)
import jax
import jax.numpy as jnp
from jax.experimental import pallas as pl


def kernel(x, edge_index, W1, b1, W2, b2, W3, b3):
    raise NotImplementedError("write your pallas kernel here")


# speedup vs baseline: 6.5082x; 6.5082x over previous
import functools

import jax
import jax.numpy as jnp
from jax import lax
from jax.experimental import pallas as pl
from jax.experimental.pallas import tpu as pltpu
from jax.experimental.pallas import tpu_sc as plsc

N = 10000
E = 320000
D = 128

NC = 2
NS = 16
NW = NC * NS
EPW = E // NW
CH = 80
NCHUNK = EPW // CH
RPS = 624
ZR = 16


def _segsum_body(h_hbm, src_hbm, dst_hbm, out_hbm, src_v, dst_v, rows_v,
                 zbuf, acc_sh, sem):
    c = lax.axis_index("c")
    s = lax.axis_index("s")
    wid = s * NC + c

    pltpu.sync_copy(src_hbm.at[wid], src_v)
    pltpu.sync_copy(dst_hbm.at[wid], dst_v)

    zero16 = jnp.zeros((16,), jnp.float32)
    for r in range(ZR):
        for k in range(D // 16):
            zbuf[r, pl.ds(k * 16, 16)] = zero16

    row0 = pl.multiple_of(s * RPS, 8)
    nz = jnp.where(s == NS - 1, (N - RPS * (NS - 1)) // ZR, RPS // ZR)

    def zcopy(m, _):
        off = pl.multiple_of(row0 + m * ZR, 8)
        pltpu.sync_copy(zbuf, acc_sh.at[pl.ds(off, ZR)])
        return 0

    lax.fori_loop(0, nz, zcopy, 0)

    plsc.subcore_barrier()

    def chunk(j, _):
        pltpu.async_copy(h_hbm.at[src_v.at[j]], rows_v, sem).wait()
        pltpu.sync_copy(rows_v, acc_sh.at[dst_v.at[j]], add=True)
        return 0

    lax.fori_loop(0, NCHUNK, chunk, 0)

    plsc.subcore_barrier()

    @pl.when(s < NS - 1)
    def _():
        pltpu.sync_copy(acc_sh.at[pl.ds(row0, RPS)],
                        out_hbm.at[c, pl.ds(row0, RPS)])

    @pl.when(s == NS - 1)
    def _():
        last0 = RPS * (NS - 1)
        pltpu.sync_copy(acc_sh.at[pl.ds(last0, N - last0)],
                        out_hbm.at[c, pl.ds(last0, N - last0)])


_segsum_sc = functools.partial(
    pl.kernel,
    _segsum_body,
    out_type=jax.ShapeDtypeStruct((NC, N, D), jnp.float32),
    mesh=plsc.VectorSubcoreMesh(core_axis_name="c", subcore_axis_name="s"),
    scratch_types=[
        pltpu.VMEM((NCHUNK, CH), jnp.int32),
        pltpu.VMEM((NCHUNK, CH), jnp.int32),
        pltpu.VMEM((CH, D), jnp.float32),
        pltpu.VMEM((ZR, D), jnp.float32),
        pltpu.VMEM_SHARED((N, D), jnp.float32),
        pltpu.SemaphoreType.DMA,
    ],
)()


def _mm_body(a0_ref, a1_ref, wt_ref, b_ref, o_ref, *, relu):
    h = a0_ref[...] + a1_ref[...]
    y = jnp.dot(h, wt_ref[...], preferred_element_type=jnp.float32)
    y = y + b_ref[...]
    if relu:
        y = jnp.maximum(y, 0.0)
    o_ref[...] = y


def _mm(a01, wt, b, relu):
    BR = 2000
    return pl.pallas_call(
        functools.partial(_mm_body, relu=relu),
        grid=(N // BR,),
        in_specs=[
            pl.BlockSpec((BR, D), lambda i: (i, 0)),
            pl.BlockSpec((BR, D), lambda i: (i, 0)),
            pl.BlockSpec((D, D), lambda i: (0, 0)),
            pl.BlockSpec((1, D), lambda i: (0, 0)),
        ],
        out_specs=pl.BlockSpec((BR, D), lambda i: (i, 0)),
        out_shape=jax.ShapeDtypeStruct((N, D), jnp.float32),
    )(a01[0], a01[1], wt, b)


def kernel(x, edge_index, W1, b1, W2, b2, W3, b3):
    dst = edge_index[0].reshape(NW, NCHUNK, CH)
    src = edge_index[1].reshape(NW, NCHUNK, CH)

    s1 = _segsum_sc(x, src, dst)
    h1 = _mm(s1, W1.T, b1.reshape(1, D), relu=True)
    s2 = _segsum_sc(h1, src, dst)
    h2 = _mm(s2, W2.T, b2.reshape(1, D), relu=True)
    s3 = _segsum_sc(h2, src, dst)
    return _mm(s3, W3.T, b3.reshape(1, D), relu=False)
